# Initial kernel scaffold; baseline (speedup 1.0000x reference)
import jax
import jax.numpy as jnp
from jax.experimental import pallas as pl


def kernel(x, node_weights, W_self, b_self, W_rel, gamma, beta, edge_index_0, edge_index_1, edge_index_2, edge_index_3):
    raise NotImplementedError("write your pallas kernel here")


# speedup vs baseline: 6.7088x; 6.7088x over previous
import jax
import jax.numpy as jnp
from jax import lax
from jax.experimental import pallas as pl
from jax.experimental.pallas import tpu as pltpu
from jax.experimental.pallas import tpu_sc as plsc

N, D, R, E = 10000, 128, 4, 80000

NC, NS = 2, 16
DA = 144
CH = 128
EPT = 5120
NCHUNK = EPT // CH
EPAD = EPT * NS
S_ROWS = 10240
RPT = S_ROWS // NS
REL_PER_SC = R // NC

_mesh = plsc.VectorSubcoreMesh(
    core_axis_name="c", subcore_axis_name="s", num_cores=NC, num_subcores=NS)


def _sc_accum_body(xa_hbm, src_hbm, tgt_hbm, zrow_hbm, agg_hbm,
                   src_v, tgt_v, rows_v, acc_sh):
    c = lax.axis_index("c")
    s = lax.axis_index("s")
    row0 = s * RPT
    for rr in range(REL_PER_SC):
        r = c * REL_PER_SC + rr
        pltpu.sync_copy(zrow_hbm, acc_sh.at[pl.ds(row0, RPT)])
        plsc.subcore_barrier()

        chunk0 = s * NCHUNK

        @pl.loop(0, NCHUNK)
        def _(k):
            ch = chunk0 + k
            pltpu.sync_copy(src_hbm.at[r, ch], src_v)
            pltpu.sync_copy(tgt_hbm.at[r, ch], tgt_v)
            pltpu.sync_copy(xa_hbm.at[src_v], rows_v)
            pltpu.sync_copy(rows_v, acc_sh.at[tgt_v], add=True)

        plsc.subcore_barrier()
        pltpu.sync_copy(acc_sh.at[pl.ds(row0, RPT)],
                        agg_hbm.at[r, pl.ds(row0, RPT)])
        if rr + 1 < REL_PER_SC:
            plsc.subcore_barrier()


_sc_accum = pl.kernel(
    _sc_accum_body,
    out_type=jax.ShapeDtypeStruct((R, S_ROWS, DA), jnp.float32),
    mesh=_mesh,
    scratch_types=[
        pltpu.VMEM((CH,), jnp.int32),
        pltpu.VMEM((CH,), jnp.int32),
        pltpu.VMEM((CH, DA), jnp.float32),
        pltpu.VMEM_SHARED((S_ROWS, DA), jnp.float32),
    ],
    compiler_params=pltpu.CompilerParams(use_tc_tiling_on_sc=False),
)


def _prep_body(x_ref, nw_ref, o_ref):
    o_ref[:, :D] = x_ref[...] * nw_ref[...]
    shp = (o_ref.shape[0], DA - D)
    o_ref[:, D:] = jnp.broadcast_to(
        (lax.broadcasted_iota(jnp.int32, shp, 1) == 0).astype(jnp.float32), shp)


def _finish_body(x_ref, ws_ref, b_ref, wr_ref, g_ref, bt_ref, agg_ref, o_ref):
    acc = jnp.dot(x_ref[...], ws_ref[...],
                  preferred_element_type=jnp.float32) + b_ref[...]
    for r in range(R):
        inv = 1.0 / jnp.maximum(agg_ref[r, :, D:D + 1], 1.0)
        m = agg_ref[r, :, :D] * inv
        acc = acc + jnp.dot(m, wr_ref[r], preferred_element_type=jnp.float32)
    h = jnp.maximum(acc, 0.0)
    mean = jnp.mean(h, axis=-1, keepdims=True)
    cent = h - mean
    var = jnp.mean(cent * cent, axis=-1, keepdims=True)
    o_ref[...] = cent * lax.rsqrt(var + 1e-5) * g_ref[...] + bt_ref[...]


def kernel(x, node_weights, W_self, b_self, W_rel, gamma, beta,
           edge_index_0, edge_index_1, edge_index_2, edge_index_3):
    pad = EPAD - E
    srcs, tgts = [], []
    for ei in (edge_index_0, edge_index_1, edge_index_2, edge_index_3):
        srcs.append(jnp.concatenate([ei[0], jnp.zeros((pad,), jnp.int32)]))
        tgts.append(jnp.concatenate([ei[1], jnp.full((pad,), N, jnp.int32)]))
    src_all = jnp.stack(srcs).reshape(R, EPAD // CH, CH)
    tgt_all = jnp.stack(tgts).reshape(R, EPAD // CH, CH)
    zrow = jnp.zeros((RPT, DA), jnp.float32)

    BLKP = 1000
    xaug = pl.pallas_call(
        _prep_body,
        grid=(N // BLKP,),
        in_specs=[
            pl.BlockSpec((BLKP, D), lambda i: (i, 0)),
            pl.BlockSpec((BLKP, 1), lambda i: (i, 0)),
        ],
        out_specs=pl.BlockSpec((BLKP, DA), lambda i: (i, 0)),
        out_shape=jax.ShapeDtypeStruct((N, DA), jnp.float32),
    )(x, node_weights.reshape(N, 1))

    agg = _sc_accum(xaug, src_all, tgt_all, zrow)

    BLK = 1000
    y = pl.pallas_call(
        _finish_body,
        grid=(N // BLK,),
        in_specs=[
            pl.BlockSpec((BLK, D), lambda i: (i, 0)),
            pl.BlockSpec((D, D), lambda i: (0, 0)),
            pl.BlockSpec((1, D), lambda i: (0, 0)),
            pl.BlockSpec((R, D, D), lambda i: (0, 0, 0)),
            pl.BlockSpec((1, D), lambda i: (0, 0)),
            pl.BlockSpec((1, D), lambda i: (0, 0)),
            pl.BlockSpec((R, BLK, DA), lambda i: (0, i, 0)),
        ],
        out_specs=pl.BlockSpec((BLK, D), lambda i: (i, 0)),
        out_shape=jax.ShapeDtypeStruct((N, D), jnp.float32),
    )(x, W_self, b_self.reshape(1, D), W_rel, gamma.reshape(1, D),
      beta.reshape(1, D), agg)
    return y


# speedup vs baseline: 15.4865x; 2.3084x over previous
import jax
import jax.numpy as jnp
from jax import lax
from jax.experimental import pallas as pl
from jax.experimental.pallas import tpu as pltpu
from jax.experimental.pallas import tpu_sc as plsc

N, D, R, E = 10000, 128, 4, 80000

NC, NS = 2, 16
DA = 144
CH = 100
EPT = E // NS
NCHUNK = EPT // CH
S_ROWS = 10016
RPT = S_ROWS // NS
REL_PER_SC = R // NC

_mesh = plsc.VectorSubcoreMesh(
    core_axis_name="c", subcore_axis_name="s", num_cores=NC, num_subcores=NS)


def _sc_accum_body(xa_hbm, src_hbm, tgt_hbm, zrow_hbm, agg_hbm,
                   src_v, tgt_v, rows0, rows1, acc_sh, gsem0, gsem1):
    c = lax.axis_index("c")
    s = lax.axis_index("s")
    row0 = s * RPT
    chunk0 = s * NCHUNK
    for rr in range(REL_PER_SC):
        r = c * REL_PER_SC + rr
        pltpu.sync_copy(zrow_hbm, acc_sh.at[pl.ds(row0, RPT)])
        pltpu.sync_copy(src_hbm.at[r, pl.ds(chunk0, NCHUNK)], src_v)
        pltpu.sync_copy(tgt_hbm.at[r, pl.ds(chunk0, NCHUNK)], tgt_v)
        plsc.subcore_barrier()

        pltpu.async_copy(xa_hbm.at[src_v.at[0]], rows0, gsem0)

        @pl.loop(0, NCHUNK, step=2)
        def _(k0):
            pltpu.async_copy(xa_hbm.at[src_v.at[k0 + 1]], rows1, gsem1)
            pltpu.make_async_copy(xa_hbm.at[src_v.at[k0]], rows0, gsem0).wait()
            pltpu.sync_copy(rows0, acc_sh.at[tgt_v.at[k0]], add=True)

            @pl.when(k0 + 2 < NCHUNK)
            def _():
                pltpu.async_copy(xa_hbm.at[src_v.at[k0 + 2]], rows0, gsem0)
            pltpu.make_async_copy(
                xa_hbm.at[src_v.at[k0 + 1]], rows1, gsem1).wait()
            pltpu.sync_copy(rows1, acc_sh.at[tgt_v.at[k0 + 1]], add=True)

        plsc.subcore_barrier()
        pltpu.sync_copy(acc_sh.at[pl.ds(row0, RPT)],
                        agg_hbm.at[r, pl.ds(row0, RPT)])
        if rr + 1 < REL_PER_SC:
            plsc.subcore_barrier()


_sc_accum = pl.kernel(
    _sc_accum_body,
    out_type=jax.ShapeDtypeStruct((R, S_ROWS, DA), jnp.float32),
    mesh=_mesh,
    scratch_types=[
        pltpu.VMEM((NCHUNK, CH), jnp.int32),
        pltpu.VMEM((NCHUNK, CH), jnp.int32),
        pltpu.VMEM((CH, DA), jnp.float32),
        pltpu.VMEM((CH, DA), jnp.float32),
        pltpu.VMEM_SHARED((S_ROWS, DA), jnp.float32),
        pltpu.SemaphoreType.DMA,
        pltpu.SemaphoreType.DMA,
    ],
    compiler_params=pltpu.CompilerParams(use_tc_tiling_on_sc=False),
)


def _prep_body(x_ref, nw_ref, o_ref):
    o_ref[:, :D] = x_ref[...] * nw_ref[...]
    shp = (o_ref.shape[0], DA - D)
    o_ref[:, D:] = jnp.broadcast_to(
        (lax.broadcasted_iota(jnp.int32, shp, 1) == 0).astype(jnp.float32), shp)


def _finish_body(x_ref, ws_ref, b_ref, wr_ref, g_ref, bt_ref, agg_ref, o_ref):
    acc = jnp.dot(x_ref[...], ws_ref[...],
                  preferred_element_type=jnp.float32) + b_ref[...]
    for r in range(R):
        inv = 1.0 / jnp.maximum(agg_ref[r, :, D:D + 1], 1.0)
        m = agg_ref[r, :, :D] * inv
        acc = acc + jnp.dot(m, wr_ref[r], preferred_element_type=jnp.float32)
    h = jnp.maximum(acc, 0.0)
    mean = jnp.mean(h, axis=-1, keepdims=True)
    cent = h - mean
    var = jnp.mean(cent * cent, axis=-1, keepdims=True)
    o_ref[...] = cent * lax.rsqrt(var + 1e-5) * g_ref[...] + bt_ref[...]


def kernel(x, node_weights, W_self, b_self, W_rel, gamma, beta,
           edge_index_0, edge_index_1, edge_index_2, edge_index_3):
    edges = jnp.stack([edge_index_0, edge_index_1, edge_index_2, edge_index_3])
    src_all = edges[:, 0].reshape(R, E // CH, CH)
    tgt_all = edges[:, 1].reshape(R, E // CH, CH)
    zrow = jnp.zeros((RPT, DA), jnp.float32)

    BLKP = 1000
    xaug = pl.pallas_call(
        _prep_body,
        grid=(N // BLKP,),
        in_specs=[
            pl.BlockSpec((BLKP, D), lambda i: (i, 0)),
            pl.BlockSpec((BLKP, 1), lambda i: (i, 0)),
        ],
        out_specs=pl.BlockSpec((BLKP, DA), lambda i: (i, 0)),
        out_shape=jax.ShapeDtypeStruct((N, DA), jnp.float32),
    )(x, node_weights.reshape(N, 1))

    agg = _sc_accum(xaug, src_all, tgt_all, zrow)

    BLK = 1000
    y = pl.pallas_call(
        _finish_body,
        grid=(N // BLK,),
        in_specs=[
            pl.BlockSpec((BLK, D), lambda i: (i, 0)),
            pl.BlockSpec((D, D), lambda i: (0, 0)),
            pl.BlockSpec((1, D), lambda i: (0, 0)),
            pl.BlockSpec((R, D, D), lambda i: (0, 0, 0)),
            pl.BlockSpec((1, D), lambda i: (0, 0)),
            pl.BlockSpec((1, D), lambda i: (0, 0)),
            pl.BlockSpec((R, BLK, DA), lambda i: (0, i, 0)),
        ],
        out_specs=pl.BlockSpec((BLK, D), lambda i: (i, 0)),
        out_shape=jax.ShapeDtypeStruct((N, D), jnp.float32),
    )(x, W_self, b_self.reshape(1, D), W_rel, gamma.reshape(1, D),
      beta.reshape(1, D), agg)
    return y


# speedup vs baseline: 19.9338x; 1.2872x over previous
import jax
import jax.numpy as jnp
from jax import lax
from jax.experimental import pallas as pl
from jax.experimental.pallas import tpu as pltpu
from jax.experimental.pallas import tpu_sc as plsc

N, D, R, E = 10000, 128, 4, 80000

NC, NS = 2, 16
CH = 128
EPT = 5120
NCHUNK = EPT // CH
HALF = NCHUNK // 2
EPAD = EPT * NS
NCROWS = EPAD // CH
FROWS = 10016
HR = 80
TOT = 10112
RPT = TOT // NS
FPT = FROWS // NS
HPT = HR // NS
REL_PER_SC = R // NC

_mesh = plsc.VectorSubcoreMesh(
    core_axis_name="c", subcore_axis_name="s", num_cores=NC, num_subcores=NS)


def _sc_accum_body(xw_hbm, s0, t0, s1, t1, s2, t2, s3, t3, zrow_hbm, hidx_hbm,
                   agg_hbm, cnt_hbm,
                   src_v, tgt_v, rows0, rows1, hist_v, hidx_v, gsem0, gsem1,
                   acc_sh):
    c = lax.axis_index("c")
    s = lax.axis_index("s")
    row0 = s * RPT
    chunk0 = s * NCHUNK
    pltpu.sync_copy(hidx_hbm, hidx_v)

    def one_relation(r, src_hbm, tgt_hbm):
        pltpu.sync_copy(zrow_hbm, acc_sh.at[pl.ds(row0, RPT)])
        pltpu.sync_copy(zrow_hbm.at[pl.ds(0, HR)], hist_v)
        plsc.subcore_barrier()

        ones16 = jnp.ones((16,), jnp.float32)

        for h in range(2):
            pltpu.sync_copy(src_hbm.at[pl.ds(chunk0 + h * HALF, HALF)], src_v)
            pltpu.sync_copy(tgt_hbm.at[pl.ds(chunk0 + h * HALF, HALF)], tgt_v)
            pltpu.async_copy(xw_hbm.at[src_v.at[0]], rows0, gsem0)

            @pl.loop(0, HALF, step=2)
            def _(k0):
                pltpu.async_copy(xw_hbm.at[src_v.at[k0 + 1]], rows1, gsem1)
                pltpu.make_async_copy(
                    xw_hbm.at[src_v.at[k0]], rows0, gsem0).wait()
                pltpu.sync_copy(rows0, acc_sh.at[tgt_v.at[k0]], add=True)
                for g in range(CH // 16):
                    t = tgt_v[k0, pl.ds(g * 16, 16)]
                    plsc.addupdate_scatter(
                        hist_v,
                        [lax.shift_right_logical(t, 7),
                         lax.bitwise_and(t, 127)], ones16)

                @pl.when(k0 + 2 < HALF)
                def _():
                    pltpu.async_copy(xw_hbm.at[src_v.at[k0 + 2]], rows0, gsem0)
                pltpu.make_async_copy(
                    xw_hbm.at[src_v.at[k0 + 1]], rows1, gsem1).wait()
                pltpu.sync_copy(rows1, acc_sh.at[tgt_v.at[k0 + 1]], add=True)
                for g in range(CH // 16):
                    t = tgt_v[k0 + 1, pl.ds(g * 16, 16)]
                    plsc.addupdate_scatter(
                        hist_v,
                        [lax.shift_right_logical(t, 7),
                         lax.bitwise_and(t, 127)], ones16)

        pltpu.sync_copy(hist_v, acc_sh.at[hidx_v], add=True)
        plsc.subcore_barrier()
        pltpu.sync_copy(acc_sh.at[pl.ds(s * FPT, FPT)],
                        agg_hbm.at[r, pl.ds(s * FPT, FPT)])
        pltpu.sync_copy(acc_sh.at[pl.ds(FROWS + s * HPT, HPT)],
                        cnt_hbm.at[r, pl.ds(s * HPT, HPT)])

    @pl.when(c == 0)
    def _():
        one_relation(0, s0, t0)
        plsc.subcore_barrier()
        one_relation(1, s1, t1)

    @pl.when(c == 1)
    def _():
        one_relation(2, s2, t2)
        plsc.subcore_barrier()
        one_relation(3, s3, t3)


_sc_accum = pl.kernel(
    _sc_accum_body,
    out_type=(
        jax.ShapeDtypeStruct((R, FROWS, D), jnp.float32),
        jax.ShapeDtypeStruct((R, HR, D), jnp.float32),
    ),
    mesh=_mesh,
    scratch_types=[
        pltpu.VMEM((HALF, CH), jnp.int32),
        pltpu.VMEM((HALF, CH), jnp.int32),
        pltpu.VMEM((CH, D), jnp.float32),
        pltpu.VMEM((CH, D), jnp.float32),
        pltpu.VMEM((HR, D), jnp.float32),
        pltpu.VMEM((HR,), jnp.int32),
        pltpu.SemaphoreType.DMA,
        pltpu.SemaphoreType.DMA,
        pltpu.VMEM_SHARED((TOT, D), jnp.float32),
    ],
    compiler_params=pltpu.CompilerParams(
        use_tc_tiling_on_sc=False, needs_layout_passes=False),
)


def _prep_body(x_ref, nw_ref, o_ref):
    o_ref[...] = x_ref[...] * nw_ref[...]


def _finish_body(x_ref, ws_ref, b_ref, wr_ref, g_ref, bt_ref, agg_ref,
                 cnt_ref, o_ref):
    acc = jnp.dot(x_ref[...], ws_ref[...],
                  preferred_element_type=jnp.float32) + b_ref[...]
    inv = 1.0 / jnp.maximum(cnt_ref[...], 1.0)
    for r in range(R):
        m = agg_ref[r] * inv[:, r:r + 1]
        acc = acc + jnp.dot(m, wr_ref[r], preferred_element_type=jnp.float32)
    h = jnp.maximum(acc, 0.0)
    mean = jnp.mean(h, axis=-1, keepdims=True)
    cent = h - mean
    var = jnp.mean(cent * cent, axis=-1, keepdims=True)
    o_ref[...] = cent * lax.rsqrt(var + 1e-5) * g_ref[...] + bt_ref[...]


def kernel(x, node_weights, W_self, b_self, W_rel, gamma, beta,
           edge_index_0, edge_index_1, edge_index_2, edge_index_3):
    pad = EPAD - E
    pad_src = (jnp.arange(pad, dtype=jnp.int32) * 61) % N
    pad_tgt = N + (jnp.arange(pad, dtype=jnp.int32) % 16)
    ei = []
    for e in (edge_index_0, edge_index_1, edge_index_2, edge_index_3):
        ei.append(jnp.concatenate([e[0], pad_src]).reshape(NCROWS, CH))
        ei.append(jnp.concatenate([e[1], pad_tgt]).reshape(NCROWS, CH))
    zrow = jnp.zeros((RPT, D), jnp.float32)
    hidx = jnp.arange(FROWS, FROWS + HR, dtype=jnp.int32)

    BLKP = 2000
    xw = pl.pallas_call(
        _prep_body,
        grid=(N // BLKP,),
        in_specs=[
            pl.BlockSpec((BLKP, D), lambda i: (i, 0)),
            pl.BlockSpec((BLKP, 1), lambda i: (i, 0)),
        ],
        out_specs=pl.BlockSpec((BLKP, D), lambda i: (i, 0)),
        out_shape=jax.ShapeDtypeStruct((N, D), jnp.float32),
    )(x, node_weights.reshape(N, 1))

    agg, cnt = _sc_accum(xw, *ei, zrow, hidx)

    cnt_t = cnt.reshape(R, HR * D).T

    BLK = 1000
    y = pl.pallas_call(
        _finish_body,
        grid=(N // BLK,),
        in_specs=[
            pl.BlockSpec((BLK, D), lambda i: (i, 0)),
            pl.BlockSpec((D, D), lambda i: (0, 0)),
            pl.BlockSpec((1, D), lambda i: (0, 0)),
            pl.BlockSpec((R, D, D), lambda i: (0, 0, 0)),
            pl.BlockSpec((1, D), lambda i: (0, 0)),
            pl.BlockSpec((1, D), lambda i: (0, 0)),
            pl.BlockSpec((R, BLK, D), lambda i: (0, i, 0)),
            pl.BlockSpec((BLK, R), lambda i: (i, 0)),
        ],
        out_specs=pl.BlockSpec((BLK, D), lambda i: (i, 0)),
        out_shape=jax.ShapeDtypeStruct((N, D), jnp.float32),
    )(x, W_self, b_self.reshape(1, D), W_rel, gamma.reshape(1, D),
      beta.reshape(1, D), agg, cnt_t)
    return y


# speedup vs baseline: 20.0544x; 1.0061x over previous
import jax
import jax.numpy as jnp
from jax import lax
from jax.experimental import pallas as pl
from jax.experimental.pallas import tpu as pltpu
from jax.experimental.pallas import tpu_sc as plsc

N, D, R, E = 10000, 128, 4, 80000

NC, NS = 2, 16
CH = 128
EPT = 5120
NCHUNK = EPT // CH
HALF = NCHUNK // 2
EPAD = EPT * NS
NCROWS = EPAD // CH
FROWS = 10016
HR = 80
TOT = 10112
RPT = TOT // NS
FPT = FROWS // NS
HPT = HR // NS
REL_PER_SC = R // NC

_mesh = plsc.VectorSubcoreMesh(
    core_axis_name="c", subcore_axis_name="s", num_cores=NC, num_subcores=NS)


def _sc_accum_body(xw_hbm, s0, t0, s1, t1, s2, t2, s3, t3, zrow_hbm, hidx_hbm,
                   agg_hbm, cnt_hbm,
                   src_v, tgt_v, rows0, rows1, hist_v, hidx_v, gsem0, gsem1,
                   acc_sh):
    c = lax.axis_index("c")
    s = lax.axis_index("s")
    row0 = s * RPT
    chunk0 = s * NCHUNK
    pltpu.sync_copy(hidx_hbm, hidx_v)

    def one_relation(r, src_hbm, tgt_hbm):
        pltpu.sync_copy(zrow_hbm, acc_sh.at[pl.ds(row0, RPT)])
        pltpu.sync_copy(zrow_hbm.at[pl.ds(0, HR)], hist_v)
        plsc.subcore_barrier()

        ones16 = jnp.ones((16,), jnp.float32)

        for h in range(2):
            pltpu.sync_copy(src_hbm.at[pl.ds(chunk0 + h * HALF, HALF)], src_v)
            pltpu.sync_copy(tgt_hbm.at[pl.ds(chunk0 + h * HALF, HALF)], tgt_v)
            pltpu.async_copy(xw_hbm.at[src_v.at[0]], rows0, gsem0)

            @pl.loop(0, HALF, step=2)
            def _(k0):
                pltpu.async_copy(xw_hbm.at[src_v.at[k0 + 1]], rows1, gsem1)
                pltpu.make_async_copy(
                    xw_hbm.at[src_v.at[k0]], rows0, gsem0).wait()
                pltpu.sync_copy(rows0, acc_sh.at[tgt_v.at[k0]], add=True)
                for g in range(CH // 16):
                    t = tgt_v[k0, pl.ds(g * 16, 16)]
                    plsc.addupdate_scatter(
                        hist_v,
                        [lax.shift_right_logical(t, 7),
                         lax.bitwise_and(t, 127)], ones16)

                @pl.when(k0 + 2 < HALF)
                def _():
                    pltpu.async_copy(xw_hbm.at[src_v.at[k0 + 2]], rows0, gsem0)
                pltpu.make_async_copy(
                    xw_hbm.at[src_v.at[k0 + 1]], rows1, gsem1).wait()
                pltpu.sync_copy(rows1, acc_sh.at[tgt_v.at[k0 + 1]], add=True)
                for g in range(CH // 16):
                    t = tgt_v[k0 + 1, pl.ds(g * 16, 16)]
                    plsc.addupdate_scatter(
                        hist_v,
                        [lax.shift_right_logical(t, 7),
                         lax.bitwise_and(t, 127)], ones16)

        pltpu.sync_copy(hist_v, acc_sh.at[hidx_v], add=True)
        plsc.subcore_barrier()
        pltpu.sync_copy(acc_sh.at[pl.ds(s * FPT, FPT)],
                        agg_hbm.at[r, pl.ds(s * FPT, FPT)])
        pltpu.sync_copy(acc_sh.at[pl.ds(FROWS + s * HPT, HPT)],
                        cnt_hbm.at[r, pl.ds(s * HPT, HPT)])

    @pl.when(c == 0)
    def _():
        one_relation(0, s0, t0)
        plsc.subcore_barrier()
        one_relation(1, s1, t1)

    @pl.when(c == 1)
    def _():
        one_relation(2, s2, t2)
        plsc.subcore_barrier()
        one_relation(3, s3, t3)


_sc_accum = pl.kernel(
    _sc_accum_body,
    out_type=(
        jax.ShapeDtypeStruct((R, FROWS, D), jnp.float32),
        jax.ShapeDtypeStruct((R, HR, D), jnp.float32),
    ),
    mesh=_mesh,
    scratch_types=[
        pltpu.VMEM((HALF, CH), jnp.int32),
        pltpu.VMEM((HALF, CH), jnp.int32),
        pltpu.VMEM((CH, D), jnp.float32),
        pltpu.VMEM((CH, D), jnp.float32),
        pltpu.VMEM((HR, D), jnp.float32),
        pltpu.VMEM((HR,), jnp.int32),
        pltpu.SemaphoreType.DMA,
        pltpu.SemaphoreType.DMA,
        pltpu.VMEM_SHARED((TOT, D), jnp.float32),
    ],
    compiler_params=pltpu.CompilerParams(
        use_tc_tiling_on_sc=False, needs_layout_passes=False),
)


def _prep_body(x_ref, nw_ref, o_ref):
    o_ref[...] = x_ref[...] * nw_ref[...]


def _self_body(x_ref, ws_ref, b_ref, o_ref):
    o_ref[...] = jnp.dot(x_ref[...], ws_ref[...],
                         preferred_element_type=jnp.float32) + b_ref[...]


def _finish_body(self_ref, wr_ref, g_ref, bt_ref, agg_ref, cnt_ref, o_ref):
    acc = self_ref[...]
    inv = 1.0 / jnp.maximum(cnt_ref[...], 1.0)
    for r in range(R):
        m = agg_ref[r] * inv[:, r:r + 1]
        acc = acc + jnp.dot(m, wr_ref[r], preferred_element_type=jnp.float32)
    h = jnp.maximum(acc, 0.0)
    mean = jnp.mean(h, axis=-1, keepdims=True)
    cent = h - mean
    var = jnp.mean(cent * cent, axis=-1, keepdims=True)
    o_ref[...] = cent * lax.rsqrt(var + 1e-5) * g_ref[...] + bt_ref[...]


def kernel(x, node_weights, W_self, b_self, W_rel, gamma, beta,
           edge_index_0, edge_index_1, edge_index_2, edge_index_3):
    pad = EPAD - E
    pad_src = (jnp.arange(pad, dtype=jnp.int32) * 61) % N
    pad_tgt = N + (jnp.arange(pad, dtype=jnp.int32) % 16)
    ei = []
    for e in (edge_index_0, edge_index_1, edge_index_2, edge_index_3):
        ei.append(jnp.concatenate([e[0], pad_src]).reshape(NCROWS, CH))
        ei.append(jnp.concatenate([e[1], pad_tgt]).reshape(NCROWS, CH))
    zrow = jnp.zeros((RPT, D), jnp.float32)
    hidx = jnp.arange(FROWS, FROWS + HR, dtype=jnp.int32)

    BLKP = 2000
    xw = pl.pallas_call(
        _prep_body,
        grid=(N // BLKP,),
        in_specs=[
            pl.BlockSpec((BLKP, D), lambda i: (i, 0)),
            pl.BlockSpec((BLKP, 1), lambda i: (i, 0)),
        ],
        out_specs=pl.BlockSpec((BLKP, D), lambda i: (i, 0)),
        out_shape=jax.ShapeDtypeStruct((N, D), jnp.float32),
    )(x, node_weights.reshape(N, 1))

    agg, cnt = _sc_accum(xw, *ei, zrow, hidx)

    BLKS = 2000
    out_self = pl.pallas_call(
        _self_body,
        grid=(N // BLKS,),
        in_specs=[
            pl.BlockSpec((BLKS, D), lambda i: (i, 0)),
            pl.BlockSpec((D, D), lambda i: (0, 0)),
            pl.BlockSpec((1, D), lambda i: (0, 0)),
        ],
        out_specs=pl.BlockSpec((BLKS, D), lambda i: (i, 0)),
        out_shape=jax.ShapeDtypeStruct((N, D), jnp.float32),
    )(x, W_self, b_self.reshape(1, D))

    cnt_t = cnt.reshape(R, HR * D).T

    BLK = 2000
    y = pl.pallas_call(
        _finish_body,
        grid=(N // BLK,),
        in_specs=[
            pl.BlockSpec((BLK, D), lambda i: (i, 0)),
            pl.BlockSpec((R, D, D), lambda i: (0, 0, 0)),
            pl.BlockSpec((1, D), lambda i: (0, 0)),
            pl.BlockSpec((1, D), lambda i: (0, 0)),
            pl.BlockSpec((R, BLK, D), lambda i: (0, i, 0)),
            pl.BlockSpec((BLK, R), lambda i: (i, 0)),
        ],
        out_specs=pl.BlockSpec((BLK, D), lambda i: (i, 0)),
        out_shape=jax.ShapeDtypeStruct((N, D), jnp.float32),
    )(out_self, W_rel, gamma.reshape(1, D),
      beta.reshape(1, D), agg, cnt_t)
    return y
